# Initial kernel scaffold; baseline (speedup 1.0000x reference)
import jax
import jax.numpy as jnp
from jax.experimental import pallas as pl


def kernel(x, tables):
    raise NotImplementedError("write your pallas kernel here")


# speedup vs baseline: 2.8661x; 2.8661x over previous
import jax
import jax.numpy as jnp
from jax import lax
from jax.experimental import pallas as pl
from jax.experimental.pallas import tpu as pltpu
from jax.experimental.pallas import tpu_sc as plsc

F = 9
V = 150
D = 128
NC = 2
NS = 16
NW = NC * NS
C = 64
K = 49
RPW = C * K
NPAD = NW * RPW
G = NPAD // C


def _body(idx_hbm, tab_hbm, out_hbm, idx_v, gbuf, acc, sem):
    wid = lax.axis_index("s") * NC + lax.axis_index("c")

    def chunk(k, carry):
        g = wid * K + k
        pltpu.sync_copy(idx_hbm.at[g], idx_v)
        handles = [
            pltpu.async_copy(tab_hbm.at[idx_v.at[f]], gbuf.at[f], sem)
            for f in range(F)
        ]
        for h in handles:
            h.wait()

        def row(r, rcarry):
            for cb in range(D // 16):
                s = pl.ds(cb * 16, 16)
                v = gbuf[0, r, s]
                for f in range(1, F):
                    v = v + gbuf[f, r, s]
                acc[r, s] = v
            return rcarry

        lax.fori_loop(0, C, row, 0)
        pltpu.sync_copy(acc, out_hbm.at[pl.ds(g * C, C)])
        return carry

    lax.fori_loop(0, K, chunk, 0)


def kernel(x, tables):
    n = x.shape[0]
    x32 = x.astype(jnp.int32)
    off = jnp.arange(F, dtype=jnp.int32) * V
    xp = jnp.pad(x32 + off[None, :], ((0, NPAD - n), (0, 0)))
    idx3 = xp.reshape(G, C, F).transpose(0, 2, 1)
    tab_flat = tables.reshape(F * V, D)

    run = pl.kernel(
        _body,
        out_type=jax.ShapeDtypeStruct((NPAD, D), jnp.float32),
        mesh=plsc.VectorSubcoreMesh(core_axis_name="c", subcore_axis_name="s"),
        scratch_types=[
            pltpu.VMEM((F, C), jnp.int32),
            pltpu.VMEM((F, C, D), jnp.float32),
            pltpu.VMEM((C, D), jnp.float32),
            pltpu.SemaphoreType.DMA,
        ],
    )
    out = run(idx3, tab_flat)
    return out[:n]
